# Initial kernel scaffold; baseline (speedup 1.0000x reference)
import functools

import jax
import jax.numpy as jnp
from jax import lax
from jax.experimental import pallas as pl
from jax.experimental.pallas import tpu as pltpu
from jax.experimental.pallas import tpu_sc as plsc

NC = 2
NS = 16
LANES = 16
NW = NC * NS

G = 128
K = 4
C = G * K


@functools.partial(jax.jit, static_argnums=(4, 5))
def _embed(token_table, seg_plus, seq2d, lab_flat, N, E):
    npw = N // NW
    n_chunks = npw // C

    mesh = plsc.VectorSubcoreMesh(
        core_axis_name="c", subcore_axis_name="s", num_cores=NC, num_subcores=NS
    )

    @functools.partial(
        pl.kernel,
        out_type=jax.ShapeDtypeStruct((N, E), jnp.float32),
        mesh=mesh,
        scratch_types=[
            pltpu.VMEM((K, G), jnp.int32),
            pltpu.VMEM((C,), jnp.int32),
            pltpu.VMEM((C, E), jnp.float32),
            pltpu.VMEM((4, E), jnp.float32),
            pltpu.SemaphoreType.DMA,
        ],
    )
    def k(tok_hbm, seg_hbm, seq_hbm, lab_hbm, out_hbm, idx_v, lab_v, rows_v, seg_v, sem):
        wid = lax.axis_index("s") * NC + lax.axis_index("c")
        base0 = wid * npw
        pltpu.sync_copy(seg_hbm, seg_v)

        def chunk_body(i, carry):
            base = base0 + i * C
            pltpu.sync_copy(seq_hbm.at[pl.ds(base // G, K)], idx_v)
            pltpu.sync_copy(lab_hbm.at[pl.ds(base, C)], lab_v)
            for j in range(K):
                pltpu.async_copy(
                    tok_hbm.at[idx_v.at[j]],
                    rows_v.at[pl.ds(j * G, G)],
                    sem,
                )
            pltpu.make_async_copy(rows_v, rows_v, sem).wait()

            def row_body(r, c2):
                lab = lab_v[r]
                for jj in range(E // LANES):
                    sl = pl.ds(jj * LANES, LANES)
                    rows_v[r, sl] = rows_v[r, sl] + seg_v[lab, sl]
                return c2

            lax.fori_loop(0, C, row_body, 0, unroll=4)
            pltpu.sync_copy(rows_v, out_hbm.at[pl.ds(base, C)])
            return carry

        lax.fori_loop(0, n_chunks, chunk_body, 0)

    return k(token_table, seg_plus, seq2d, lab_flat)


def kernel(token_table, segment_table, pe, sequence, segment_label):
    B, L = sequence.shape
    V, E = token_table.shape
    N = B * L
    pos = pe[0, L]
    seg_plus = segment_table + pos[None, :]
    seg_plus = jnp.concatenate([seg_plus, jnp.zeros((1, E), seg_plus.dtype)], 0)
    seq2d = sequence.reshape(N // G, G)
    lab_flat = segment_label.reshape(N)
    out = _embed(token_table, seg_plus, seq2d, lab_flat, N, E)
    return out.reshape(B, L, E)


# speedup vs baseline: 2.0259x; 2.0259x over previous
import functools

import jax
import jax.numpy as jnp
from jax import lax
from jax.experimental import pallas as pl
from jax.experimental.pallas import tpu as pltpu
from jax.experimental.pallas import tpu_sc as plsc

NC = 2
NS = 16
LANES = 16
NW = NC * NS

G = 128
K = 8
C = G * K


@functools.partial(jax.jit, static_argnums=(4, 5))
def _embed(token_table, seg_plus, seq2d, lab_flat, N, E):
    npw = N // NW
    n_chunks = npw // C

    mesh = plsc.VectorSubcoreMesh(
        core_axis_name="c", subcore_axis_name="s", num_cores=NC, num_subcores=NS
    )

    @functools.partial(
        pl.kernel,
        out_type=jax.ShapeDtypeStruct((N, E), jnp.float32),
        mesh=mesh,
        scratch_types=[
            pltpu.VMEM((K, G), jnp.int32),
            pltpu.VMEM((C,), jnp.int32),
            pltpu.VMEM((C, E), jnp.float32),
            pltpu.VMEM((4 * E,), jnp.float32),
            pltpu.SemaphoreType.DMA,
        ],
        compiler_params=pltpu.CompilerParams(use_tc_tiling_on_sc=False),
    )
    def k(tok_hbm, seg_hbm, seq_hbm, lab_hbm, out_hbm, idx_v, lab_v, rows_v, seg_v, sem):
        wid = lax.axis_index("s") * NC + lax.axis_index("c")
        base0 = wid * npw
        pltpu.sync_copy(seg_hbm, seg_v)

        def chunk_body(i, carry):
            base = pl.multiple_of(base0 + i * C, C)
            pltpu.sync_copy(seq_hbm.at[pl.ds(pl.multiple_of(base // G, K), K)], idx_v)
            pltpu.sync_copy(lab_hbm.at[pl.ds(base, C)], lab_v)
            descs = [
                pltpu.async_copy(
                    tok_hbm.at[idx_v.at[j]],
                    rows_v.at[pl.ds(j * G, G)],
                    sem,
                )
                for j in range(K)
            ]
            for d in descs:
                d.wait()

            def row_body(t, c2):
                labv = lab_v[pl.ds(t * LANES, LANES)]
                for kk in range(LANES):
                    lab = labv[kk]
                    r = t * LANES + kk
                    for jj in range(E // LANES):
                        sl = pl.ds(jj * LANES, LANES)
                        sv = seg_v[pl.ds(lab * E + jj * LANES, LANES)]
                        rows_v[r, sl] = rows_v[r, sl] + sv
                return c2

            lax.fori_loop(0, C // LANES, row_body, 0)
            pltpu.sync_copy(rows_v, out_hbm.at[pl.ds(base, C)])
            return carry

        lax.fori_loop(0, n_chunks, chunk_body, 0)

    return k(token_table, seg_plus, seq2d, lab_flat)


def kernel(token_table, segment_table, pe, sequence, segment_label):
    B, L = sequence.shape
    V, E = token_table.shape
    N = B * L
    pos = pe[0, L]
    seg_plus = segment_table + pos[None, :]
    seg_plus = jnp.concatenate([seg_plus, jnp.zeros((1, E), seg_plus.dtype)], 0).reshape(-1)
    seq2d = sequence.reshape(N // G, G)
    lab_flat = segment_label.reshape(N)
    out = _embed(token_table, seg_plus, seq2d, lab_flat, N, E)
    return out.reshape(B, L, E)


# speedup vs baseline: 2.0352x; 1.0046x over previous
import functools

import jax
import jax.numpy as jnp
from jax import lax
from jax.experimental import pallas as pl
from jax.experimental.pallas import tpu as pltpu
from jax.experimental.pallas import tpu_sc as plsc

NC = 2
NS = 16
LANES = 16
NW = NC * NS

G = 128
K = 8
C = G * K


@functools.partial(jax.jit, static_argnums=(3, 4))
def _embed(token_table, seg_plus, comb, N, E):
    npw = N // NW
    n_chunks = npw // C

    mesh = plsc.VectorSubcoreMesh(
        core_axis_name="c", subcore_axis_name="s", num_cores=NC, num_subcores=NS
    )

    @functools.partial(
        pl.kernel,
        out_type=jax.ShapeDtypeStruct((N, E), jnp.float32),
        mesh=mesh,
        scratch_types=[
            pltpu.VMEM((C,), jnp.int32),
            pltpu.VMEM((K, G), jnp.int32),
            pltpu.VMEM((C, E), jnp.float32),
            pltpu.VMEM((4 * E,), jnp.float32),
            pltpu.SemaphoreType.DMA,
        ],
        compiler_params=pltpu.CompilerParams(use_tc_tiling_on_sc=False),
    )
    def k(tok_hbm, seg_hbm, comb_hbm, out_hbm, comb_v, idx_v, rows_v, seg_v, sem):
        wid = lax.axis_index("s") * NC + lax.axis_index("c")
        base0 = wid * npw
        pltpu.sync_copy(seg_hbm, seg_v)

        def chunk_body(i, carry):
            base = pl.multiple_of(base0 + i * C, C)
            pltpu.sync_copy(comb_hbm.at[pl.ds(base, C)], comb_v)
            for t in range(C // LANES):
                v = comb_v[pl.ds(t * LANES, LANES)]
                idx_v[(t * LANES) // G, pl.ds((t * LANES) % G, LANES)] = (
                    lax.shift_right_logical(v, 2)
                )
            descs = [
                pltpu.async_copy(
                    tok_hbm.at[idx_v.at[j]],
                    rows_v.at[pl.ds(j * G, G)],
                    sem,
                )
                for j in range(K)
            ]
            for d in descs:
                d.wait()

            def row_body(t, c2):
                labv = lax.bitwise_and(comb_v[pl.ds(t * LANES, LANES)], 3)
                for kk in range(LANES):
                    lab = labv[kk]
                    r = t * LANES + kk
                    for jj in range(E // LANES):
                        sl = pl.ds(jj * LANES, LANES)
                        sv = seg_v[pl.ds(lab * E + jj * LANES, LANES)]
                        rows_v[r, sl] = rows_v[r, sl] + sv
                return c2

            lax.fori_loop(0, C // LANES, row_body, 0)
            pltpu.sync_copy(rows_v, out_hbm.at[pl.ds(base, C)])
            return carry

        lax.fori_loop(0, n_chunks, chunk_body, 0)

    return k(token_table, seg_plus, comb)


def kernel(token_table, segment_table, pe, sequence, segment_label):
    B, L = sequence.shape
    V, E = token_table.shape
    N = B * L
    pos = pe[0, L]
    seg_plus = segment_table + pos[None, :]
    seg_plus = jnp.concatenate([seg_plus, jnp.zeros((1, E), seg_plus.dtype)], 0).reshape(-1)
    comb = (sequence * 4 + segment_label).reshape(N)
    out = _embed(token_table, seg_plus, comb, N, E)
    return out.reshape(B, L, E)


# speedup vs baseline: 2.1594x; 1.0611x over previous
import functools

import jax
import jax.numpy as jnp
from jax import lax
from jax.experimental import pallas as pl
from jax.experimental.pallas import tpu as pltpu
from jax.experimental.pallas import tpu_sc as plsc

NC = 2
NS = 16
LANES = 16
NW = NC * NS

G = 128
K = 4
C = G * K


@functools.partial(jax.jit, static_argnums=(3, 4))
def _embed(token_table, seg_plus, comb, N, E):
    npw = N // NW
    n_chunks = npw // C
    assert n_chunks % 2 == 0
    EV = E // LANES

    mesh = plsc.VectorSubcoreMesh(
        core_axis_name="c", subcore_axis_name="s", num_cores=NC, num_subcores=NS
    )

    @functools.partial(
        pl.kernel,
        out_type=jax.ShapeDtypeStruct((N, E), jnp.float32),
        mesh=mesh,
        scratch_types=[
            pltpu.VMEM((npw,), jnp.int32),
            pltpu.VMEM((K, G), jnp.int32),
            pltpu.VMEM((K, G), jnp.int32),
            pltpu.VMEM((C, E), jnp.float32),
            pltpu.VMEM((C, E), jnp.float32),
            pltpu.VMEM((4 * E,), jnp.float32),
            pltpu.SemaphoreType.DMA,
            pltpu.SemaphoreType.DMA,
            pltpu.SemaphoreType.DMA,
            pltpu.SemaphoreType.DMA,
        ],
        compiler_params=pltpu.CompilerParams(use_tc_tiling_on_sc=False),
    )
    def k(tok_hbm, seg_hbm, comb_hbm, out_hbm,
          comb_v, idx0, idx1, rows0, rows1, seg_v,
          sg0, sg1, sw0, sw1):
        idx_b = (idx0, idx1)
        rows_b = (rows0, rows1)
        sg_b = (sg0, sg1)
        sw_b = (sw0, sw1)
        wid = lax.axis_index("s") * NC + lax.axis_index("c")
        base0 = pl.multiple_of(wid * npw, C)
        pltpu.sync_copy(seg_hbm, seg_v)
        pltpu.sync_copy(comb_hbm.at[pl.ds(base0, npw)], comb_v)

        def decode(i, buf):
            off = i * C
            for t in range(C // LANES):
                v = comb_v[pl.ds(off + t * LANES, LANES)]
                idx_b[buf][(t * LANES) // G, pl.ds((t * LANES) % G, LANES)] = (
                    lax.shift_right_logical(v, 2)
                )

        def fire_gathers(buf, i):
            base = pl.multiple_of(base0 + i * C, C)
            for j in range(K):
                pltpu.async_copy(
                    tok_hbm.at[idx_b[buf].at[j]],
                    rows_b[buf].at[pl.ds(j * G, G)],
                    sg_b[buf],
                )

        def drain(sem, ref):
            pltpu.make_async_copy(tok_hbm.at[pl.ds(0, C)], ref, sem).wait()

        def seg_add(i, buf):
            rows = rows_b[buf]
            off = i * C

            def row_body(t, c2):
                labv = comb_v[pl.ds(off + t * LANES, LANES)]
                for kk in range(LANES):
                    soff = lax.bitwise_and(labv[kk], 3) * E
                    r = t * LANES + kk
                    for jj in range(EV):
                        sl = pl.ds(jj * LANES, LANES)
                        sv = seg_v[pl.ds(soff + jj * LANES, LANES)]
                        rows[r, sl] = rows[r, sl] + sv
                return c2

            lax.fori_loop(0, C // LANES, row_body, 0)

        def fire_write(i, buf):
            base = pl.multiple_of(base0 + i * C, C)
            pltpu.async_copy(rows_b[buf], out_hbm.at[pl.ds(base, C)], sw_b[buf])

        decode(0, 0)
        fire_gathers(0, 0)

        def pair_body(t, carry):
            i = t * 2
            decode(i + 1, 1)

            @pl.when(t > 0)
            def _():
                drain(sw_b[1], rows_b[1])
            fire_gathers(1, i + 1)
            drain(sg_b[0], rows_b[0])
            seg_add(i, 0)
            fire_write(i, 0)

            @pl.when(t < n_chunks // 2 - 1)
            def _():
                decode(i + 2, 0)
                drain(sw_b[0], rows_b[0])
                fire_gathers(0, i + 2)
            drain(sg_b[1], rows_b[1])
            seg_add(i + 1, 1)
            fire_write(i + 1, 1)
            return carry

        lax.fori_loop(0, n_chunks // 2, pair_body, 0)
        drain(sw_b[0], rows_b[0])
        drain(sw_b[1], rows_b[1])

    return k(token_table, seg_plus, comb)


def kernel(token_table, segment_table, pe, sequence, segment_label):
    B, L = sequence.shape
    V, E = token_table.shape
    N = B * L
    pos = pe[0, L]
    seg_plus = segment_table + pos[None, :]
    seg_plus = jnp.concatenate([seg_plus, jnp.zeros((1, E), seg_plus.dtype)], 0).reshape(-1)
    comb = (sequence * 4 + segment_label).reshape(N)
    out = _embed(token_table, seg_plus, comb, N, E)
    return out.reshape(B, L, E)


# speedup vs baseline: 2.1625x; 1.0014x over previous
import functools

import jax
import jax.numpy as jnp
from jax import lax
from jax.experimental import pallas as pl
from jax.experimental.pallas import tpu as pltpu
from jax.experimental.pallas import tpu_sc as plsc

NC = 2
NS = 16
LANES = 16
NW = NC * NS

G = 128
K = 2
C = G * K


@functools.partial(jax.jit, static_argnums=(3, 4))
def _embed(token_table, seg_plus, comb, N, E):
    npw = N // NW
    n_chunks = npw // C
    assert n_chunks % 2 == 0
    EV = E // LANES

    mesh = plsc.VectorSubcoreMesh(
        core_axis_name="c", subcore_axis_name="s", num_cores=NC, num_subcores=NS
    )

    @functools.partial(
        pl.kernel,
        out_type=jax.ShapeDtypeStruct((N, E), jnp.float32),
        mesh=mesh,
        scratch_types=[
            pltpu.VMEM((npw,), jnp.int32),
            pltpu.VMEM((K, G), jnp.int32),
            pltpu.VMEM((K, G), jnp.int32),
            pltpu.VMEM((C, 2 * E), jnp.float32),
            pltpu.VMEM((C, 2 * E), jnp.float32),
            pltpu.VMEM((C, E), jnp.float32),
            pltpu.VMEM((C, E), jnp.float32),
            pltpu.VMEM((4 * E,), jnp.float32),
            pltpu.SemaphoreType.DMA,
            pltpu.SemaphoreType.DMA,
            pltpu.SemaphoreType.DMA,
            pltpu.SemaphoreType.DMA,
        ],
        compiler_params=pltpu.CompilerParams(use_tc_tiling_on_sc=False),
    )
    def k(tok_hbm, seg_hbm, comb_hbm, out_hbm,
          comb_v, idx0, idx1, rowsg0, rowsg1, rows0, rows1, seg_v,
          sg0, sg1, sw0, sw1):
        rowsg_b = (rowsg0, rowsg1)
        idx_b = (idx0, idx1)
        rows_b = (rows0, rows1)
        sg_b = (sg0, sg1)
        sw_b = (sw0, sw1)
        wid = lax.axis_index("s") * NC + lax.axis_index("c")
        base0 = pl.multiple_of(wid * npw, C)
        pltpu.sync_copy(seg_hbm, seg_v)
        pltpu.sync_copy(comb_hbm.at[pl.ds(base0, npw)], comb_v)

        def decode(i, buf):
            off = i * C
            for t in range(C // LANES):
                v = comb_v[pl.ds(off + t * LANES, LANES)]
                idx_b[buf][(t * LANES) // G, pl.ds((t * LANES) % G, LANES)] = (
                    lax.shift_right_logical(v, 2)
                )

        def fire_gathers(buf, i):
            base = pl.multiple_of(base0 + i * C, C)
            for j in range(K):
                pltpu.async_copy(
                    tok_hbm.at[idx_b[buf].at[j]],
                    rowsg_b[buf].at[pl.ds(j * G, G)],
                    sg_b[buf],
                )

        def drain_g(sem, ref):
            pltpu.make_async_copy(tok_hbm.at[pl.ds(0, C)], ref, sem).wait()

        def drain(sem, ref):
            pltpu.make_async_copy(out_hbm.at[pl.ds(0, C)], ref, sem).wait()

        def seg_add(i, buf):
            rows = rows_b[buf]
            rowsg = rowsg_b[buf]
            off = i * C

            def row_body(t, c2):
                labv = comb_v[pl.ds(off + t * LANES, LANES)]
                for kk in range(LANES):
                    soff = lax.bitwise_and(labv[kk], 3) * E
                    r = t * LANES + kk
                    for jj in range(EV):
                        sl = pl.ds(jj * LANES, LANES)
                        sv = seg_v[pl.ds(soff + jj * LANES, LANES)]
                        rows[r, sl] = rowsg[r, sl] + sv
                return c2

            lax.fori_loop(0, C // LANES, row_body, 0)

        def fire_write(i, buf):
            base = pl.multiple_of(base0 + i * C, C)
            pltpu.async_copy(rows_b[buf], out_hbm.at[pl.ds(base, C)], sw_b[buf])

        decode(0, 0)
        fire_gathers(0, 0)

        def pair_body(t, carry):
            i = t * 2
            decode(i + 1, 1)

            @pl.when(t > 0)
            def _():
                drain(sw_b[1], rows_b[1])
            fire_gathers(1, i + 1)
            drain_g(sg_b[0], rowsg_b[0])
            seg_add(i, 0)
            fire_write(i, 0)

            @pl.when(t < n_chunks // 2 - 1)
            def _():
                decode(i + 2, 0)
                drain(sw_b[0], rows_b[0])
                fire_gathers(0, i + 2)
            drain_g(sg_b[1], rowsg_b[1])
            seg_add(i + 1, 1)
            fire_write(i + 1, 1)
            return carry

        lax.fori_loop(0, n_chunks // 2, pair_body, 0)
        drain(sw_b[0], rows_b[0])
        drain(sw_b[1], rows_b[1])

    return k(token_table, seg_plus, comb)


def kernel(token_table, segment_table, pe, sequence, segment_label):
    B, L = sequence.shape
    V, E = token_table.shape
    N = B * L
    pos = pe[0, L]
    seg_plus = segment_table + pos[None, :]
    seg_plus = jnp.concatenate([seg_plus, jnp.zeros((1, E), seg_plus.dtype)], 0).reshape(-1)
    comb = (sequence * 4 + segment_label).reshape(N)
    token_table = jnp.pad(token_table, ((0, 0), (0, E)))
    out = _embed(token_table, seg_plus, comb, N, E)
    return out.reshape(B, L, E)
